# baseline (speedup 1.0000x reference)
import jax
import jax.numpy as jnp
from jax import lax
from jax.experimental import pallas as pl
from jax.experimental.pallas import tpu as pltpu
from jax.experimental.pallas import tpu_sc as plsc

NUM_SEGMENTS = 256
N_ROWS = 50000
D = 512
NC = 2
NS = 16
DC = D // NC
B = 64
NBF = N_ROWS // B
REM = N_ROWS - NBF * B
FULL_ITERS = NBF // NS
PAIRS = FULL_ITERS // 2
TAIL = NBF - FULL_ITERS * NS
SEGS_PER_TILE = NUM_SEGMENTS // NS
L = 16
NJ = DC // L
ACC = NUM_SEGMENTS * DC
CNTW = NUM_SEGMENTS * L


def _body(h_hbm, ids_hbm, out_hbm,
          hb0, hb1, id0, id1, hrem, idrem, acc, cnt, regbuf, cregbuf,
          rb0, rb1, rc0, rc1, comb, ccomb,
          outb, sem0, sem1, isem0, isem1, rsem0, rsem1, csem0, csem1,
          part_hbm, cpart_hbm):
    core = lax.axis_index("c")
    sid = lax.axis_index("s")
    col0 = core * DC
    hbufs = (hb0, hb1)
    ibufs = (id0, id1)
    sems = (sem0, sem1)
    isems = (isem0, isem1)
    rbufs = (rb0, rb1)
    rcnts = (rc0, rc1)
    rsems = (rsem0, rsem1)
    csems = (csem0, csem1)

    zero16 = jnp.zeros((L,), jnp.float32)
    one16 = jnp.ones((L,), jnp.float32)

    def zstep(g, carry):
        for u in range(L):
            acc[pl.ds((g * L + u) * L, L)] = zero16
        return carry
    lax.fori_loop(0, ACC // (L * L), zstep, 0)
    for g in range(CNTW // L):
        cnt[pl.ds(g * L, L)] = zero16

    def copies(b, k):
        i = pltpu.make_async_copy(ids_hbm.at[pl.ds(b * B, B)], ibufs[k], isems[k])
        h = pltpu.make_async_copy(
            h_hbm.at[pl.ds(b * B, B), pl.ds(col0, DC)], hbufs[k], sems[k])
        return i, h

    def start(b, k):
        i, h = copies(b, k)
        i.start()
        h.start()

    def wait(b, k):
        i, h = copies(b, k)
        i.wait()
        h.wait()

    def flush(cur_id):
        base = cur_id * DC
        for jb in range(NJ // 4):
            cols = [(jb * 4 + j) * L for j in range(4)]
            vals = [acc[pl.ds(base + c, L)] + regbuf[pl.ds(c, L)] for c in cols]
            for v, c in zip(vals, cols):
                acc[pl.ds(base + c, L)] = v
        cnt[pl.ds(cur_id * L, L)] += cregbuf[pl.ds(0, L)]

    def row_direct(hbuf, r, idr):
        base = idr * DC
        for jb in range(NJ // 4):
            cols = [(jb * 4 + j) * L for j in range(4)]
            vals = [acc[pl.ds(base + c, L)] + hbuf[r, pl.ds(c, L)] for c in cols]
            for v, c in zip(vals, cols):
                acc[pl.ds(base + c, L)] = v
        cnt[pl.ds(idr * L, L)] += one16

    def process(hbuf, ibuf, ngroups, cur_id0):
        def gstep(g, cur_id):
            idvec = ibuf[pl.ds(g * L, L)]
            first = idvec[0]
            last = idvec[L - 1]
            same = jnp.logical_and(first == cur_id, last == cur_id)

            @pl.when(same)
            def _fast():
                W = 8
                for jb in range(NJ // W):
                    cols = [(jb * W + j) * L for j in range(W)]
                    regs = [regbuf[pl.ds(c, L)] for c in cols]
                    for rr in range(L):
                        r = g * L + rr
                        for j, c in enumerate(cols):
                            regs[j] = regs[j] + hbuf[r, pl.ds(c, L)]
                    for j, c in enumerate(cols):
                        regbuf[pl.ds(c, L)] = regs[j]
                cregbuf[pl.ds(0, L)] += 16.0 * one16

            @pl.when(jnp.logical_not(same))
            def _slow():
                flush(cur_id)
                for rr in range(L - 1):
                    r = g * L + rr
                    row_direct(hbuf, r, idvec[rr])
                rlast = g * L + (L - 1)
                for j in range(NJ):
                    regbuf[pl.ds(j * L, L)] = hbuf[rlast, pl.ds(j * L, L)]
                cregbuf[pl.ds(0, L)] = one16

            return jnp.where(same, cur_id, last)

        return lax.fori_loop(0, ngroups, gstep, cur_id0)

    for j in range(NJ):
        regbuf[pl.ds(j * L, L)] = zero16
    cregbuf[pl.ds(0, L)] = zero16
    start(sid, 0)
    start(NS + sid, 1)

    def pair_step(p, cur_id):
        wait((2 * p) * NS + sid, 0)
        cur_id = process(hb0, id0, B // L, cur_id)

        @pl.when(p < PAIRS - 1)
        def _():
            start((2 * p + 2) * NS + sid, 0)

        wait((2 * p + 1) * NS + sid, 1)
        cur_id = process(hb1, id1, B // L, cur_id)

        @pl.when(p < PAIRS - 1)
        def _():
            start((2 * p + 3) * NS + sid, 1)
        return cur_id

    final_id = lax.fori_loop(0, PAIRS, pair_step, jnp.int32(0))
    flush(final_id)

    @pl.when(sid < TAIL)
    def _extra():
        b = FULL_ITERS * NS + sid
        start(b, 0)
        wait(b, 0)

        def estep(g, carry):
            idvec = id0[pl.ds(g * L, L)]
            for rr in range(L):
                row_direct(hb0, g * L + rr, idvec[rr])
            return carry
        lax.fori_loop(0, B // L, estep, 0)

    @pl.when(sid == NS - 1)
    def _rem():
        rows = pl.ds(NBF * B, REM)
        pltpu.sync_copy(ids_hbm.at[rows], idrem)
        pltpu.sync_copy(h_hbm.at[rows, pl.ds(col0, DC)], hrem)
        idvec = idrem[pl.ds(0, L)]
        for rr in range(L):
            row_direct(hrem, rr, idvec[rr])

    pltpu.sync_copy(acc, part_hbm.at[core, sid])
    pltpu.sync_copy(cnt, cpart_hbm.at[core, sid])
    plsc.subcore_barrier()

    seg0 = sid * SEGS_PER_TILE
    for g in range(SEGS_PER_TILE * DC // (L * L)):
        for u in range(L):
            comb[pl.ds((g * L + u) * L, L)] = zero16
    for g in range(SEGS_PER_TILE):
        ccomb[pl.ds(g * L, L)] = zero16

    def rcopies(t, k):
        a = pltpu.make_async_copy(
            part_hbm.at[core, t, pl.ds(seg0 * DC, SEGS_PER_TILE * DC)],
            rbufs[k], rsems[k])
        c = pltpu.make_async_copy(
            cpart_hbm.at[core, t, pl.ds(seg0 * L, SEGS_PER_TILE * L)],
            rcnts[k], csems[k])
        return a, c

    def rstart(t, k):
        a, c = rcopies(t, k)
        a.start()
        c.start()

    def rwait(t, k):
        a, c = rcopies(t, k)
        a.wait()
        c.wait()

    def add_slab(rb, rc):
        def astep(g, c2):
            for u in range(0, L, 4):
                os = [(g * L + u + z) * L for z in range(4)]
                vals = [comb[pl.ds(o, L)] + rb[pl.ds(o, L)] for o in os]
                for v, o in zip(vals, os):
                    comb[pl.ds(o, L)] = v
            return c2
        lax.fori_loop(0, SEGS_PER_TILE * DC // (L * L), astep, 0)
        for g in range(SEGS_PER_TILE):
            ccomb[pl.ds(g * L, L)] += rc[pl.ds(g * L, L)]

    rstart(0, 0)

    def comb_pair(q, carry):
        t0 = 2 * q
        rwait(t0, 0)
        rstart(t0 + 1, 1)
        add_slab(rbufs[0], rcnts[0])
        rwait(t0 + 1, 1)

        @pl.when(q < NS // 2 - 1)
        def _():
            rstart(t0 + 2, 0)
        add_slab(rbufs[1], rcnts[1])
        return carry

    lax.fori_loop(0, NS // 2, comb_pair, 0)

    for s in range(SEGS_PER_TILE):
        recip = 1.0 / jnp.maximum(ccomb[pl.ds(s * L, L)], 1.0)
        for j in range(NJ):
            outb[s, pl.ds(j * L, L)] = comb[pl.ds(s * DC + j * L, L)] * recip
    pltpu.sync_copy(outb, out_hbm.at[pl.ds(seg0, SEGS_PER_TILE), pl.ds(col0, DC)])


@jax.jit
def _seg_mean(h, ids):
    mesh = plsc.VectorSubcoreMesh(
        core_axis_name="c", subcore_axis_name="s", num_cores=NC, num_subcores=NS
    )
    k = pl.kernel(
        _body,
        out_type=jax.ShapeDtypeStruct((NUM_SEGMENTS, D), jnp.float32),
        mesh=mesh,
        compiler_params=pltpu.CompilerParams(use_tc_tiling_on_sc=True),
        scratch_types=[
            pltpu.VMEM((B, DC), jnp.float32),
            pltpu.VMEM((B, DC), jnp.float32),
            pltpu.VMEM((B,), jnp.int32),
            pltpu.VMEM((B,), jnp.int32),
            pltpu.VMEM((REM, DC), jnp.float32),
            pltpu.VMEM((REM,), jnp.int32),
            pltpu.VMEM((ACC,), jnp.float32),
            pltpu.VMEM((CNTW,), jnp.float32),
            pltpu.VMEM((DC,), jnp.float32),
            pltpu.VMEM((L,), jnp.float32),
            pltpu.VMEM((SEGS_PER_TILE * DC,), jnp.float32),
            pltpu.VMEM((SEGS_PER_TILE * DC,), jnp.float32),
            pltpu.VMEM((SEGS_PER_TILE * L,), jnp.float32),
            pltpu.VMEM((SEGS_PER_TILE * L,), jnp.float32),
            pltpu.VMEM((SEGS_PER_TILE * DC,), jnp.float32),
            pltpu.VMEM((SEGS_PER_TILE * L,), jnp.float32),
            pltpu.VMEM((SEGS_PER_TILE, DC), jnp.float32),
            pltpu.SemaphoreType.DMA,
            pltpu.SemaphoreType.DMA,
            pltpu.SemaphoreType.DMA,
            pltpu.SemaphoreType.DMA,
            pltpu.SemaphoreType.DMA,
            pltpu.SemaphoreType.DMA,
            pltpu.SemaphoreType.DMA,
            pltpu.SemaphoreType.DMA,
            pltpu.HBM((NC, NS, ACC), jnp.float32),
            pltpu.HBM((NC, NS, CNTW), jnp.float32),
        ],
    )
    return k(h, ids)


def kernel(h, graph_ids):
    return _seg_mean(h, graph_ids.astype(jnp.int32))


# speedup vs baseline: 1.1073x; 1.1073x over previous
import functools

import jax
import jax.numpy as jnp
from jax import lax
from jax.experimental import pallas as pl
from jax.experimental.pallas import tpu as pltpu
from jax.experimental.pallas import tpu_sc as plsc

NUM_SEGMENTS = 256
N_ROWS = 50000
D = 512
NC = 2
NS = 16
DC = D // NC
B = 125
NB = N_ROWS // B
BLOCKS_PER_TILE = NB // NS
SEGS_PER_TILE = NUM_SEGMENTS // NS
L = 16


def _seg_mean_body(h_hbm, ids_hbm, out_hbm,
                   hblk0, hblk1, ids_v, ones_v, zer_v, zcnt_v,
                   sums_v, cnt_v, out_v, sem0, sem1,
                   sums_sh, cnt_sh):
    core = lax.axis_index("c")
    sid = lax.axis_index("s")
    col0 = core * DC
    hbufs = (hblk0, hblk1)
    sems = (sem0, sem1)

    zero16 = jnp.zeros((L,), jnp.float32)
    one16 = jnp.ones((L,), jnp.float32)
    for r in range(B):
        ones_v[r, :] = one16
    for r in range(SEGS_PER_TILE):
        zcnt_v[r, :] = zero16
        for j in range(DC // L):
            zer_v[r, pl.ds(j * L, L)] = zero16

    seg0 = sid * SEGS_PER_TILE
    b0 = sid * BLOCKS_PER_TILE
    pltpu.sync_copy(zer_v, sums_sh.at[pl.ds(seg0, SEGS_PER_TILE)])
    pltpu.sync_copy(zcnt_v, cnt_sh.at[pl.ds(seg0, SEGS_PER_TILE)])
    pltpu.sync_copy(ids_hbm.at[pl.ds(b0, BLOCKS_PER_TILE)], ids_v)
    plsc.subcore_barrier()

    def gather(i, buf, sem):
        rows = pl.ds((b0 + i) * B, B)
        return pltpu.async_copy(h_hbm.at[rows, pl.ds(col0, DC)], buf, sem)

    pending = gather(0, hbufs[0], sems[0])
    for i in range(BLOCKS_PER_TILE):
        cur = i % 2
        pending.wait()
        if i + 1 < BLOCKS_PER_TILE:
            pending = gather(i + 1, hbufs[1 - cur], sems[1 - cur])
        pltpu.sync_copy(hbufs[cur], sums_sh.at[ids_v.at[i]], add=True)
        pltpu.sync_copy(ones_v, cnt_sh.at[ids_v.at[i]], add=True)
    plsc.subcore_barrier()

    pltpu.sync_copy(sums_sh.at[pl.ds(seg0, SEGS_PER_TILE)], sums_v)
    pltpu.sync_copy(cnt_sh.at[pl.ds(seg0, SEGS_PER_TILE)], cnt_v)
    for r in range(SEGS_PER_TILE):
        recip = 1.0 / jnp.maximum(cnt_v[r, :], 1.0)
        for j in range(DC // L):
            out_v[r, pl.ds(j * L, L)] = sums_v[r, pl.ds(j * L, L)] * recip
    pltpu.sync_copy(out_v, out_hbm.at[pl.ds(seg0, SEGS_PER_TILE), pl.ds(col0, DC)])


@jax.jit
def _seg_mean(h, ids2d):
    mesh = plsc.VectorSubcoreMesh(
        core_axis_name="c", subcore_axis_name="s", num_cores=NC, num_subcores=NS
    )
    k = pl.kernel(
        _seg_mean_body,
        out_type=jax.ShapeDtypeStruct((NUM_SEGMENTS, D), jnp.float32),
        mesh=mesh,
        compiler_params=pltpu.CompilerParams(use_tc_tiling_on_sc=False),
        scratch_types=[
            pltpu.VMEM((B, DC), jnp.float32),
            pltpu.VMEM((B, DC), jnp.float32),
            pltpu.VMEM((BLOCKS_PER_TILE, B), jnp.int32),
            pltpu.VMEM((B, L), jnp.float32),
            pltpu.VMEM((SEGS_PER_TILE, DC), jnp.float32),
            pltpu.VMEM((SEGS_PER_TILE, L), jnp.float32),
            pltpu.VMEM((SEGS_PER_TILE, DC), jnp.float32),
            pltpu.VMEM((SEGS_PER_TILE, L), jnp.float32),
            pltpu.VMEM((SEGS_PER_TILE, DC), jnp.float32),
            pltpu.SemaphoreType.DMA,
            pltpu.SemaphoreType.DMA,
            pltpu.VMEM_SHARED((NUM_SEGMENTS, DC), jnp.float32),
            pltpu.VMEM_SHARED((NUM_SEGMENTS, L), jnp.float32),
        ],
    )
    return k(h, ids2d)


def kernel(h, graph_ids):
    ids2d = graph_ids.astype(jnp.int32).reshape(NB, B)
    return _seg_mean(h, ids2d)
